# baseline (speedup 1.0000x reference)
import functools

import jax
import jax.numpy as jnp
from jax import lax
from jax.experimental import pallas as pl
from jax.experimental.pallas import tpu as pltpu
from jax.experimental.pallas import tpu_sc as plsc

_N = 10000
_E = 320000
_NC = 2
_NS = 16
_NW = _NC * _NS
_CK = 128
_NCHUNK = _E // _CK
_CPT = -(-_NCHUNK // _NW)
_CLAST = _NCHUNK - (_NW - 1) * _CPT
_RPT = 624
_RLAST = _N - 15 * _RPT


def _wid_and_nch():
    c = lax.axis_index("c")
    s = lax.axis_index("s")
    wid = s * _NC + c
    nch = jnp.where(wid == _NW - 1, jnp.int32(_CLAST), jnp.int32(_CPT))
    return c, s, wid, nch


def _each_span(fn):
    s = lax.axis_index("s")

    @pl.when(s < _NS - 1)
    def _():
        fn(s * _RPT, _RPT)

    @pl.when(s == _NS - 1)
    def _():
        fn(15 * _RPT, _RLAST)


def _each_piece(fn):
    s = lax.axis_index("s")

    @pl.when(s < _NS - 1)
    def _():
        for o, ln in ((0, 160), (160, 160), (320, 160), (480, 144)):
            fn(s * _RPT + o, ln)

    @pl.when(s == _NS - 1)
    def _():
        for o in (0, 160, 320, 480):
            fn(15 * _RPT + o, 160)


def _load_my_chunks(src2d, dst, wid):
    @pl.when(wid < _NW - 1)
    def _():
        pltpu.sync_copy(src2d.at[pl.ds(wid * _CPT, _CPT)], dst)

    @pl.when(wid == _NW - 1)
    def _():
        pltpu.sync_copy(src2d.at[pl.ds((_NW - 1) * _CPT, _CLAST)],
                        dst.at[pl.ds(0, _CLAST)])


_CPT2 = -(-_NCHUNK // _NS)
_C2LAST = _NCHUNK - 15 * _CPT2
_HCH = _NCHUNK // _NC
_CPT3 = -(-_HCH // _NS)
_C3LAST = _HCH - 15 * _CPT3


@functools.partial(
    pl.kernel,
    out_type=[jax.ShapeDtypeStruct((_NCHUNK, _CK), jnp.float32),
              jax.ShapeDtypeStruct((_N,), jnp.float32)],
    mesh=plsc.VectorSubcoreMesh(core_axis_name="c", subcore_axis_name="s"),
    compiler_params=pltpu.CompilerParams(use_tc_tiling_on_sc=False),
    scratch_types=[
        pltpu.VMEM((_CPT2, _CK), jnp.int32),
        pltpu.VMEM((_CPT2, _CK), jnp.float32),
        pltpu.VMEM((_CPT3, _CK), jnp.int32),
        pltpu.VMEM((_RLAST,), jnp.float32),
        pltpu.VMEM((_RLAST,), jnp.float32),
        pltpu.VMEM((8, _CK), jnp.float32),
        pltpu.VMEM((8, _CK), jnp.float32),
        pltpu.VMEM((2, 8, _CK), jnp.float32),
        pltpu.SemaphoreType.DMA((8,)),
        pltpu.SemaphoreType.DMA((8,)),
        pltpu.VMEM_SHARED((_N,), jnp.float32),
        pltpu.VMEM_SHARED((_N,), jnp.float32),
    ],
)
def _sc_norm(row_hbm, col_hbm, ew_hbm, wp_hbm, d2_hbm,
             cidx_v, ew_v, ridx_v, stage_v, stage2_v, dr_v, dc_v, wp_v,
             dsem, esem, acc_sh, dinv_sh):
    c = lax.axis_index("c")
    s = lax.axis_index("s")
    n2 = jnp.where(s == _NS - 1, jnp.int32(_C2LAST), jnp.int32(_CPT2))

    @pl.when(s < _NS - 1)
    def _():
        pltpu.sync_copy(col_hbm.at[pl.ds(s * _CPT2, _CPT2)], cidx_v)
        pltpu.sync_copy(ew_hbm.at[pl.ds(s * _CPT2, _CPT2)], ew_v)

    @pl.when(s == _NS - 1)
    def _():
        pltpu.sync_copy(col_hbm.at[pl.ds(15 * _CPT2, _C2LAST)],
                        cidx_v.at[pl.ds(0, _C2LAST)])
        pltpu.sync_copy(ew_hbm.at[pl.ds(15 * _CPT2, _C2LAST)],
                        ew_v.at[pl.ds(0, _C2LAST)])

    def fill(i, carry):
        stage_v[pl.ds(i * 16, 16)] = jnp.zeros((16,), jnp.float32)
        return carry

    lax.fori_loop(jnp.int32(0), jnp.int32(_RLAST // 16), fill, jnp.int32(0))

    def zero(off, ln):
        pltpu.sync_copy(stage_v.at[pl.ds(0, ln)], acc_sh.at[pl.ds(off, ln)])

    _each_span(zero)
    plsc.subcore_barrier()

    def superstep(u, carry):
        for j in range(8):
            t = u * 8 + j

            @pl.when(t < n2)
            def _():
                pltpu.async_copy(ew_v.at[t], acc_sh.at[cidx_v.at[t]],
                                 dsem.at[jnp.int32(j)], add=True)

        for j in range(8):
            t = u * 8 + j

            @pl.when(t < n2)
            def _():
                pltpu.make_async_copy(ew_v.at[t], acc_sh.at[cidx_v.at[t]],
                                      dsem.at[jnp.int32(j)]).wait()

        return carry

    lax.fori_loop(jnp.int32(0), jnp.int32(-(-_CPT2 // 8)), superstep,
                  jnp.int32(0))
    plsc.subcore_barrier()

    def dinv_span(off, ln):
        pltpu.sync_copy(acc_sh.at[pl.ds(off, ln)], stage_v.at[pl.ds(0, ln)])

        def conv_body(i, carry):
            deg = stage_v[pl.ds(i * 16, 16)] + 1.0
            bits = lax.bitcast_convert_type(deg, jnp.int32)
            seed = jnp.int32(0x5F3759DF) - lax.shift_right_logical(
                bits, jnp.int32(1))
            y = lax.bitcast_convert_type(seed, jnp.float32)
            half = 0.5 * deg
            for _ in range(3):
                y = y * (1.5 - half * y * y)
            dinv = y
            stage2_v[pl.ds(i * 16, 16)] = dinv
            stage_v[pl.ds(i * 16, 16)] = dinv * dinv
            return carry

        lax.fori_loop(jnp.int32(0), jnp.int32(ln // 16), conv_body,
                      jnp.int32(0))
        pltpu.sync_copy(stage2_v.at[pl.ds(0, ln)],
                        dinv_sh.at[pl.ds(off, ln)])

        @pl.when(c == 0)
        def _():
            pltpu.sync_copy(stage_v.at[pl.ds(0, ln)],
                            d2_hbm.at[pl.ds(off, ln)])

    _each_span(dinv_span)

    g3 = c * _HCH + s * _CPT3
    n3 = jnp.where(s == _NS - 1, jnp.int32(_C3LAST), jnp.int32(_CPT3))

    @pl.when(s < _NS - 1)
    def _():
        pltpu.sync_copy(row_hbm.at[pl.ds(g3, _CPT3)], ridx_v)
        pltpu.sync_copy(col_hbm.at[pl.ds(g3, _CPT3)],
                        cidx_v.at[pl.ds(0, _CPT3)])
        pltpu.sync_copy(ew_hbm.at[pl.ds(g3, _CPT3)],
                        ew_v.at[pl.ds(0, _CPT3)])

    @pl.when(s == _NS - 1)
    def _():
        pltpu.sync_copy(row_hbm.at[pl.ds(g3, _C3LAST)],
                        ridx_v.at[pl.ds(0, _C3LAST)])
        pltpu.sync_copy(col_hbm.at[pl.ds(g3, _C3LAST)],
                        cidx_v.at[pl.ds(0, _C3LAST)])
        pltpu.sync_copy(ew_hbm.at[pl.ds(g3, _C3LAST)],
                        ew_v.at[pl.ds(0, _C3LAST)])

    plsc.subcore_barrier()

    def batch_pair(u, carry):
        for par in range(2):
            bt = u * 2 + par

            for j in range(8):
                t = bt * 8 + j

                @pl.when(t < n3)
                def _():
                    pltpu.async_copy(dinv_sh.at[ridx_v.at[t]],
                                     dr_v.at[jnp.int32(j)],
                                     dsem.at[jnp.int32(j)])
                    pltpu.async_copy(dinv_sh.at[cidx_v.at[t]],
                                     dc_v.at[jnp.int32(j)],
                                     esem.at[jnp.int32(j)])

            for j in range(8):
                t = bt * 8 + j

                @pl.when(t < n3)
                def _():
                    pltpu.make_async_copy(dinv_sh.at[ridx_v.at[t]],
                                          dr_v.at[jnp.int32(j)],
                                          dsem.at[jnp.int32(j)]).wait()
                    pltpu.make_async_copy(dinv_sh.at[cidx_v.at[t]],
                                          dc_v.at[jnp.int32(j)],
                                          esem.at[jnp.int32(j)]).wait()

            for j in range(8):
                t = bt * 8 + j

                @pl.when(t < n3)
                def _():
                    def wp_body(g, gcarry):
                        sl = pl.ds(g * 16, 16)
                        wp_v[par, j, sl] = (ew_v[t, sl] * dr_v[j, sl]
                                            * dc_v[j, sl])
                        return gcarry

                    lax.fori_loop(jnp.int32(0), jnp.int32(_CK // 16),
                                  wp_body, jnp.int32(0))

            @pl.when(bt * 8 + 8 <= n3)
            def _():
                pltpu.sync_copy(wp_v.at[jnp.int32(par)],
                                wp_hbm.at[pl.ds(g3 + bt * 8, 8)])

            for j in range(8):
                t = bt * 8 + j

                @pl.when((t < n3) & (bt * 8 + 8 > n3))
                def _():
                    pltpu.sync_copy(wp_v.at[jnp.int32(par), jnp.int32(j)],
                                    wp_hbm.at[g3 + t])

        return carry

    lax.fori_loop(jnp.int32(0), jnp.int32(-(-_CPT3 // 16)), batch_pair,
                  jnp.int32(0))


def _make_sc_conv(d):
    nvreg = d // 16

    @functools.partial(
        pl.kernel,
        out_type=jax.ShapeDtypeStruct((_NC * _N, d), jnp.float32),
        mesh=plsc.VectorSubcoreMesh(core_axis_name="c", subcore_axis_name="s"),
        compiler_params=pltpu.CompilerParams(use_tc_tiling_on_sc=False),
        scratch_types=[
            pltpu.VMEM((_CPT, _CK), jnp.int32),
            pltpu.VMEM((_CPT, _CK), jnp.int32),
            pltpu.VMEM((_CPT, _CK), jnp.float32),
            pltpu.VMEM((3, _CK, d), jnp.float32),
            pltpu.VMEM((160, d), jnp.float32),
            pltpu.SemaphoreType.DMA((3,)),
            pltpu.SemaphoreType.DMA((3,)),
            pltpu.VMEM_SHARED((_N, d), jnp.float32),
        ],
    )
    def _sc_conv(row_hbm, col_hbm, ew_hbm, p_hbm, out_hbm,
                 ridx_v, cidx_v, ew_v, rows_v, stage_v, gsem, ssem, acc_sh):
        c, s, wid, nch = _wid_and_nch()

        _load_my_chunks(row_hbm, ridx_v, wid)
        _load_my_chunks(col_hbm, cidx_v, wid)
        _load_my_chunks(ew_hbm, ew_v, wid)

        def fill(i, carry):
            for f in range(nvreg):
                stage_v[i, pl.ds(f * 16, 16)] = jnp.zeros((16,), jnp.float32)
            return carry

        lax.fori_loop(jnp.int32(0), jnp.int32(160), fill, jnp.int32(0))

        def zero(off, ln):
            pltpu.sync_copy(stage_v.at[pl.ds(0, ln)],
                            acc_sh.at[pl.ds(off, ln)])

        _each_piece(zero)
        plsc.subcore_barrier()

        def start_gather(t, b):
            b = jnp.int32(b)
            pltpu.async_copy(p_hbm.at[ridx_v.at[t]], rows_v.at[b],
                             gsem.at[b])

        def wait_gather(t, b):
            b = jnp.int32(b)
            pltpu.make_async_copy(p_hbm.at[ridx_v.at[t]], rows_v.at[b],
                                  gsem.at[b]).wait()

        def start_scatter(t, b):
            b = jnp.int32(b)
            pltpu.async_copy(rows_v.at[b], acc_sh.at[cidx_v.at[t]],
                             ssem.at[b], add=True)

        def wait_scatter(b):
            b = jnp.int32(b)
            pltpu.make_async_copy(rows_v.at[b], acc_sh.at[cidx_v.at[jnp.int32(0)]],
                                  ssem.at[b]).wait()

        start_gather(jnp.int32(0), 0)

        def tri_body(u, carry):
            for k in range(3):
                t = u * 3 + k
                b = k
                nb = (k + 1) % 3

                @pl.when(t < nch)
                def _():
                    @pl.when(t + 1 < nch)
                    def _():
                        @pl.when(t >= 2)
                        def _():
                            wait_scatter(nb)

                        start_gather(t + 1, nb)

                    wait_gather(t, b)

                    def g_body(g, gcarry):
                        ewg = ew_v[t, pl.ds(g * 16, 16)]
                        for j in range(16):
                            e = g * 16 + j
                            wv = jnp.full((16,), ewg[j], jnp.float32)
                            for f in range(nvreg):
                                rows_v[b, e, pl.ds(f * 16, 16)] = (
                                    rows_v[b, e, pl.ds(f * 16, 16)] * wv)
                        return gcarry

                    lax.fori_loop(jnp.int32(0), jnp.int32(_CK // 16), g_body,
                                  jnp.int32(0))
                    start_scatter(t, b)

            return carry

        lax.fori_loop(jnp.int32(0), jnp.int32(-(-_CPT // 3)), tri_body,
                      jnp.int32(0))
        wait_scatter(0)
        wait_scatter(1)
        wait_scatter(2)
        plsc.subcore_barrier()

        def writeout(off, ln):
            pltpu.sync_copy(acc_sh.at[pl.ds(off, ln)],
                            stage_v.at[pl.ds(0, ln)])
            pltpu.sync_copy(stage_v.at[pl.ds(0, ln)],
                            out_hbm.at[pl.ds(c * _N + off, ln)])

        _each_piece(writeout)

    return _sc_conv


_sc_conv64 = _make_sc_conv(64)
_sc_conv16 = _make_sc_conv(16)


def _tc_mm1_body(x_ref, w1_ref, mm_ref):
    mm_ref[...] = jnp.dot(x_ref[...], w1_ref[...],
                          preferred_element_type=jnp.float32)


def _tc_mid_body(acc_ref, mm1_ref, d2_ref, b1_ref, w2_ref, p2_ref):
    d2 = d2_ref[...][:, None]
    a = acc_ref[0] + acc_ref[1] + d2 * mm1_ref[...]
    h = jnp.maximum(a + b1_ref[...], 0.0)
    p2_ref[...] = jnp.dot(h, w2_ref[...], preferred_element_type=jnp.float32)


def _tc_last_body(acc_ref, mm2_ref, d2_ref, b2_ref, out_ref):
    a = acc_ref[0] + acc_ref[1] + d2_ref[...][:, None] * mm2_ref[...]
    out_ref[...] = a + b2_ref[...]


def kernel(x, edge_index, edge_weight, W1, b1, W2, b2):
    n, nfeat = x.shape
    nhid = W1.shape[1]
    ncls = W2.shape[1]

    row = edge_index[0].astype(jnp.int32).reshape(_NCHUNK, _CK)
    col = edge_index[1].astype(jnp.int32).reshape(_NCHUNK, _CK)
    ew2d = edge_weight.astype(jnp.float32).reshape(_NCHUNK, _CK)
    xf = x.astype(jnp.float32)

    wp, d2 = _sc_norm(row, col, ew2d)
    mm1 = pl.pallas_call(
        _tc_mm1_body,
        out_shape=jax.ShapeDtypeStruct((n, nhid), jnp.float32),
    )(xf, W1.astype(jnp.float32))

    acc1 = _sc_conv64(row, col, wp, mm1).reshape(_NC, n, nhid)

    p2 = pl.pallas_call(
        _tc_mid_body,
        out_shape=jax.ShapeDtypeStruct((n, ncls), jnp.float32),
    )(acc1, mm1, d2, b1.astype(jnp.float32), W2.astype(jnp.float32))

    acc2 = _sc_conv16(row, col, wp, p2).reshape(_NC, n, ncls)

    out = pl.pallas_call(
        _tc_last_body,
        out_shape=jax.ShapeDtypeStruct((n, ncls), jnp.float32),
    )(acc2, p2, d2, b2.astype(jnp.float32))

    return out.astype(x.dtype if x.dtype == jnp.float64 else jnp.float64)


# speedup vs baseline: 1.0983x; 1.0983x over previous
import functools

import jax
import jax.numpy as jnp
from jax import lax
from jax.experimental import pallas as pl
from jax.experimental.pallas import tpu as pltpu
from jax.experimental.pallas import tpu_sc as plsc

_N = 10000
_E = 320000
_NC = 2
_NS = 16
_NW = _NC * _NS
_CK = 128
_NCHUNK = _E // _CK
_CPT = -(-_NCHUNK // _NW)
_CLAST = _NCHUNK - (_NW - 1) * _CPT
_RPT = 624
_RLAST = _N - 15 * _RPT


def _wid_and_nch():
    c = lax.axis_index("c")
    s = lax.axis_index("s")
    wid = s * _NC + c
    nch = jnp.where(wid == _NW - 1, jnp.int32(_CLAST), jnp.int32(_CPT))
    return c, s, wid, nch


def _each_span(fn):
    s = lax.axis_index("s")

    @pl.when(s < _NS - 1)
    def _():
        fn(s * _RPT, _RPT)

    @pl.when(s == _NS - 1)
    def _():
        fn(15 * _RPT, _RLAST)


def _each_piece(fn):
    s = lax.axis_index("s")

    @pl.when(s < _NS - 1)
    def _():
        for o, ln in ((0, 160), (160, 160), (320, 160), (480, 144)):
            fn(s * _RPT + o, ln)

    @pl.when(s == _NS - 1)
    def _():
        for o in (0, 160, 320, 480):
            fn(15 * _RPT + o, 160)


def _load_my_chunks(src2d, dst, wid):
    @pl.when(wid < _NW - 1)
    def _():
        pltpu.sync_copy(src2d.at[pl.ds(wid * _CPT, _CPT)], dst)

    @pl.when(wid == _NW - 1)
    def _():
        pltpu.sync_copy(src2d.at[pl.ds((_NW - 1) * _CPT, _CLAST)],
                        dst.at[pl.ds(0, _CLAST)])


@functools.partial(
    pl.kernel,
    out_type=jax.ShapeDtypeStruct((_NC * _N,), jnp.float32),
    mesh=plsc.VectorSubcoreMesh(core_axis_name="c", subcore_axis_name="s"),
    compiler_params=pltpu.CompilerParams(use_tc_tiling_on_sc=False),
    scratch_types=[
        pltpu.VMEM((_CPT, _CK), jnp.int32),
        pltpu.VMEM((_CPT, _CK), jnp.float32),
        pltpu.VMEM((_RLAST,), jnp.float32),
        pltpu.SemaphoreType.DMA((8,)),
        pltpu.VMEM_SHARED((_N,), jnp.float32),
    ],
)
def _sc_degree(col_hbm, ew_hbm, out_hbm, cidx_v, ew_v, stage_v, sem, acc_sh):
    c, s, wid, nch = _wid_and_nch()

    _load_my_chunks(col_hbm, cidx_v, wid)
    _load_my_chunks(ew_hbm, ew_v, wid)

    def fill(i, carry):
        stage_v[pl.ds(i * 16, 16)] = jnp.zeros((16,), jnp.float32)
        return carry

    lax.fori_loop(jnp.int32(0), jnp.int32(_RLAST // 16), fill, jnp.int32(0))

    def zero(off, ln):
        pltpu.sync_copy(stage_v.at[pl.ds(0, ln)], acc_sh.at[pl.ds(off, ln)])

    _each_span(zero)
    plsc.subcore_barrier()

    def superstep(u, carry):
        for j in range(8):
            t = u * 8 + j

            @pl.when(t < nch)
            def _():
                pltpu.async_copy(ew_v.at[t], acc_sh.at[cidx_v.at[t]],
                                 sem.at[jnp.int32(j)], add=True)

        for j in range(8):
            t = u * 8 + j

            @pl.when(t < nch)
            def _():
                pltpu.make_async_copy(ew_v.at[t], acc_sh.at[cidx_v.at[t]],
                                      sem.at[jnp.int32(j)]).wait()

        return carry

    lax.fori_loop(jnp.int32(0), jnp.int32(-(-_CPT // 8)), superstep,
                  jnp.int32(0))
    plsc.subcore_barrier()

    def writeout(off, ln):
        pltpu.sync_copy(acc_sh.at[pl.ds(off, ln)], stage_v.at[pl.ds(0, ln)])
        pltpu.sync_copy(stage_v.at[pl.ds(0, ln)],
                        out_hbm.at[pl.ds(c * _N + off, ln)])

    _each_span(writeout)


def _make_sc_conv(d):
    nvreg = d // 16

    @functools.partial(
        pl.kernel,
        out_type=jax.ShapeDtypeStruct((_NC * _N, d), jnp.float32),
        mesh=plsc.VectorSubcoreMesh(core_axis_name="c", subcore_axis_name="s"),
        compiler_params=pltpu.CompilerParams(use_tc_tiling_on_sc=False),
        scratch_types=[
            pltpu.VMEM((_CPT, _CK), jnp.int32),
            pltpu.VMEM((_CPT, _CK), jnp.int32),
            pltpu.VMEM((_CPT * _CK,), jnp.float32),
            pltpu.VMEM((4, _CK, d), jnp.float32),
            pltpu.VMEM((160, d), jnp.float32),
            pltpu.SemaphoreType.DMA((4,)),
            pltpu.SemaphoreType.DMA((4,)),
            pltpu.VMEM_SHARED((_N, d), jnp.float32),
        ],
    )
    def _sc_conv(row_hbm, col_hbm, ew_hbm, p_hbm, out_hbm,
                 ridx_v, cidx_v, ew_v, rows_v, stage_v, gsem, ssem, acc_sh):
        c, s, wid, nch = _wid_and_nch()

        _load_my_chunks(row_hbm, ridx_v, wid)
        _load_my_chunks(col_hbm, cidx_v, wid)

        @pl.when(wid < _NW - 1)
        def _():
            pltpu.sync_copy(ew_hbm.at[pl.ds(wid * _CPT * _CK, _CPT * _CK)],
                            ew_v)

        @pl.when(wid == _NW - 1)
        def _():
            pltpu.sync_copy(
                ew_hbm.at[pl.ds((_NW - 1) * _CPT * _CK, _CLAST * _CK)],
                ew_v.at[pl.ds(0, _CLAST * _CK)])

        def fill(i, carry):
            for f in range(nvreg):
                stage_v[i, pl.ds(f * 16, 16)] = jnp.zeros((16,), jnp.float32)
            return carry

        lax.fori_loop(jnp.int32(0), jnp.int32(160), fill, jnp.int32(0))

        def zero(off, ln):
            pltpu.sync_copy(stage_v.at[pl.ds(0, ln)],
                            acc_sh.at[pl.ds(off, ln)])

        _each_piece(zero)
        plsc.subcore_barrier()

        def start_gather(t, b):
            b = jnp.int32(b)
            pltpu.async_copy(p_hbm.at[ridx_v.at[t]], rows_v.at[b],
                             gsem.at[b])

        def wait_gather(t, b):
            b = jnp.int32(b)
            pltpu.make_async_copy(p_hbm.at[ridx_v.at[t]], rows_v.at[b],
                                  gsem.at[b]).wait()

        def start_scatter(t, b):
            b = jnp.int32(b)
            pltpu.async_copy(rows_v.at[b], acc_sh.at[cidx_v.at[t]],
                             ssem.at[b], add=True)

        def wait_scatter(b):
            b = jnp.int32(b)
            pltpu.make_async_copy(rows_v.at[b], acc_sh.at[cidx_v.at[jnp.int32(0)]],
                                  ssem.at[b]).wait()

        start_gather(jnp.int32(0), 0)

        def tri_body(u, carry):
            for k in range(4):
                t = u * 4 + k
                b = k
                nb = (k + 1) % 4

                @pl.when(t < nch)
                def _():
                    @pl.when(t + 1 < nch)
                    def _():
                        @pl.when(t >= 3)
                        def _():
                            wait_scatter(nb)

                        start_gather(t + 1, nb)

                    wait_gather(t, b)

                    def g_body(g, gcarry):
                        ewg = ew_v[pl.ds(t * _CK + g * 16, 16)]
                        for j in range(16):
                            e = g * 16 + j
                            wv = jnp.full((16,), ewg[j], jnp.float32)
                            for f in range(nvreg):
                                rows_v[b, e, pl.ds(f * 16, 16)] = (
                                    rows_v[b, e, pl.ds(f * 16, 16)] * wv)
                        return gcarry

                    lax.fori_loop(jnp.int32(0), jnp.int32(_CK // 16), g_body,
                                  jnp.int32(0))
                    start_scatter(t, b)

            return carry

        lax.fori_loop(jnp.int32(0), jnp.int32(-(-_CPT // 4)), tri_body,
                      jnp.int32(0))
        wait_scatter(0)
        wait_scatter(1)
        wait_scatter(2)
        wait_scatter(3)
        plsc.subcore_barrier()

        def writeout(off, ln):
            pltpu.sync_copy(acc_sh.at[pl.ds(off, ln)],
                            stage_v.at[pl.ds(0, ln)])
            pltpu.sync_copy(stage_v.at[pl.ds(0, ln)],
                            out_hbm.at[pl.ds(c * _N + off, ln)])

        _each_piece(writeout)

    return _sc_conv


_sc_conv64 = _make_sc_conv(64)
_sc_conv16 = _make_sc_conv(16)


def _tc_first_body(deg_ref, x_ref, w1_ref, p1_ref, dinv_ref):
    deg = deg_ref[0] + deg_ref[1] + 1.0
    dinv = jnp.where(deg > 0, lax.rsqrt(deg), 0.0)
    dinv_ref[...] = dinv[:, None]
    mm = jnp.dot(x_ref[...], w1_ref[...], preferred_element_type=jnp.float32)
    p1_ref[...] = mm * dinv[:, None]


def _tc_mid_body(acc_ref, p1_ref, dinv_ref, b1_ref, w2_ref, p2_ref):
    dinv = dinv_ref[...]
    a = acc_ref[0] + acc_ref[1] + p1_ref[...]
    h = jnp.maximum(dinv * a + b1_ref[...], 0.0)
    p2_ref[...] = jnp.dot(h, w2_ref[...],
                          preferred_element_type=jnp.float32) * dinv


def _tc_last_body(acc_ref, p2_ref, dinv_ref, b2_ref, out_ref):
    a = acc_ref[0] + acc_ref[1] + p2_ref[...]
    out_ref[...] = dinv_ref[...] * a + b2_ref[...]


def kernel(x, edge_index, edge_weight, W1, b1, W2, b2):
    n, nfeat = x.shape
    nhid = W1.shape[1]
    ncls = W2.shape[1]

    row = edge_index[0].astype(jnp.int32).reshape(_NCHUNK, _CK)
    col = edge_index[1].astype(jnp.int32).reshape(_NCHUNK, _CK)
    ew = edge_weight.astype(jnp.float32)
    ew2d = ew.reshape(_NCHUNK, _CK)
    xf = x.astype(jnp.float32)

    deg_parts = _sc_degree(col, ew2d).reshape(_NC, n)

    p1, dinv = pl.pallas_call(
        _tc_first_body,
        out_shape=[jax.ShapeDtypeStruct((n, nhid), jnp.float32),
                   jax.ShapeDtypeStruct((n, 1), jnp.float32)],
    )(deg_parts, xf, W1.astype(jnp.float32))

    acc1 = _sc_conv64(row, col, ew, p1).reshape(_NC, n, nhid)

    p2 = pl.pallas_call(
        _tc_mid_body,
        out_shape=jax.ShapeDtypeStruct((n, ncls), jnp.float32),
    )(acc1, p1, dinv, b1.astype(jnp.float32), W2.astype(jnp.float32))

    acc2 = _sc_conv16(row, col, ew, p2).reshape(_NC, n, ncls)

    out = pl.pallas_call(
        _tc_last_body,
        out_shape=jax.ShapeDtypeStruct((n, ncls), jnp.float32),
    )(acc2, p2, dinv, b2.astype(jnp.float32))

    return out.astype(x.dtype if x.dtype == jnp.float64 else jnp.float64)


# speedup vs baseline: 1.1231x; 1.0226x over previous
import functools

import jax
import jax.numpy as jnp
from jax import lax
from jax.experimental import pallas as pl
from jax.experimental.pallas import tpu as pltpu
from jax.experimental.pallas import tpu_sc as plsc

_N = 10000
_E = 320000
_NC = 2
_NS = 16
_NW = _NC * _NS
_CK = 128
_NCHUNK = _E // _CK
_CPT = -(-_NCHUNK // _NW)
_CLAST = _NCHUNK - (_NW - 1) * _CPT
_RPT = 624
_RLAST = _N - 15 * _RPT


def _wid_and_nch():
    c = lax.axis_index("c")
    s = lax.axis_index("s")
    wid = s * _NC + c
    nch = jnp.where(wid == _NW - 1, jnp.int32(_CLAST), jnp.int32(_CPT))
    return c, s, wid, nch


def _each_span(fn):
    s = lax.axis_index("s")

    @pl.when(s < _NS - 1)
    def _():
        fn(s * _RPT, _RPT)

    @pl.when(s == _NS - 1)
    def _():
        fn(15 * _RPT, _RLAST)


def _each_piece(fn):
    s = lax.axis_index("s")

    @pl.when(s < _NS - 1)
    def _():
        for o, ln in ((0, 160), (160, 160), (320, 160), (480, 144)):
            fn(s * _RPT + o, ln)

    @pl.when(s == _NS - 1)
    def _():
        for o in (0, 160, 320, 480):
            fn(15 * _RPT + o, 160)


def _load_my_chunks(src2d, dst, wid):
    @pl.when(wid < _NW - 1)
    def _():
        pltpu.sync_copy(src2d.at[pl.ds(wid * _CPT, _CPT)], dst)

    @pl.when(wid == _NW - 1)
    def _():
        pltpu.sync_copy(src2d.at[pl.ds((_NW - 1) * _CPT, _CLAST)],
                        dst.at[pl.ds(0, _CLAST)])


@functools.partial(
    pl.kernel,
    out_type=jax.ShapeDtypeStruct((_NC * _N,), jnp.float32),
    mesh=plsc.VectorSubcoreMesh(core_axis_name="c", subcore_axis_name="s"),
    compiler_params=pltpu.CompilerParams(use_tc_tiling_on_sc=False),
    scratch_types=[
        pltpu.VMEM((_CPT, _CK), jnp.int32),
        pltpu.VMEM((_CPT, _CK), jnp.float32),
        pltpu.VMEM((_RLAST,), jnp.float32),
        pltpu.SemaphoreType.DMA((8,)),
        pltpu.VMEM_SHARED((_N,), jnp.float32),
    ],
)
def _sc_degree(col_hbm, ew_hbm, out_hbm, cidx_v, ew_v, stage_v, sem, acc_sh):
    c, s, wid, nch = _wid_and_nch()

    _load_my_chunks(col_hbm, cidx_v, wid)
    _load_my_chunks(ew_hbm, ew_v, wid)

    def fill(i, carry):
        stage_v[pl.ds(i * 16, 16)] = jnp.zeros((16,), jnp.float32)
        return carry

    lax.fori_loop(jnp.int32(0), jnp.int32(_RLAST // 16), fill, jnp.int32(0))

    def zero(off, ln):
        pltpu.sync_copy(stage_v.at[pl.ds(0, ln)], acc_sh.at[pl.ds(off, ln)])

    _each_span(zero)
    plsc.subcore_barrier()

    def superstep(u, carry):
        for j in range(8):
            t = u * 8 + j

            @pl.when(t < nch)
            def _():
                pltpu.async_copy(ew_v.at[t], acc_sh.at[cidx_v.at[t]],
                                 sem.at[jnp.int32(j)], add=True)

        for j in range(8):
            t = u * 8 + j

            @pl.when(t < nch)
            def _():
                pltpu.make_async_copy(ew_v.at[t], acc_sh.at[cidx_v.at[t]],
                                      sem.at[jnp.int32(j)]).wait()

        return carry

    lax.fori_loop(jnp.int32(0), jnp.int32(-(-_CPT // 8)), superstep,
                  jnp.int32(0))
    plsc.subcore_barrier()

    def writeout(off, ln):
        pltpu.sync_copy(acc_sh.at[pl.ds(off, ln)], stage_v.at[pl.ds(0, ln)])
        pltpu.sync_copy(stage_v.at[pl.ds(0, ln)],
                        out_hbm.at[pl.ds(c * _N + off, ln)])

    _each_span(writeout)


def _make_sc_conv(d):
    nvreg = d // 16

    @functools.partial(
        pl.kernel,
        out_type=jax.ShapeDtypeStruct((_NC * _N, d), jnp.float32),
        mesh=plsc.VectorSubcoreMesh(core_axis_name="c", subcore_axis_name="s"),
        compiler_params=pltpu.CompilerParams(use_tc_tiling_on_sc=False),
        scratch_types=[
            pltpu.VMEM((_CPT, _CK), jnp.int32),
            pltpu.VMEM((_CPT, _CK), jnp.int32),
            pltpu.VMEM((_CPT * _CK,), jnp.float32),
            pltpu.VMEM((5, _CK, d), jnp.float32),
            pltpu.VMEM((160, d), jnp.float32),
            pltpu.SemaphoreType.DMA((5,)),
            pltpu.SemaphoreType.DMA((5,)),
            pltpu.VMEM_SHARED((_N, d), jnp.float32),
        ],
    )
    def _sc_conv(row_hbm, col_hbm, ew_hbm, p_hbm, out_hbm,
                 ridx_v, cidx_v, ew_v, rows_v, stage_v, gsem, ssem, acc_sh):
        c, s, wid, nch = _wid_and_nch()

        _load_my_chunks(row_hbm, ridx_v, wid)
        _load_my_chunks(col_hbm, cidx_v, wid)

        @pl.when(wid < _NW - 1)
        def _():
            pltpu.sync_copy(ew_hbm.at[pl.ds(wid * _CPT * _CK, _CPT * _CK)],
                            ew_v)

        @pl.when(wid == _NW - 1)
        def _():
            pltpu.sync_copy(
                ew_hbm.at[pl.ds((_NW - 1) * _CPT * _CK, _CLAST * _CK)],
                ew_v.at[pl.ds(0, _CLAST * _CK)])

        def fill(i, carry):
            for f in range(nvreg):
                stage_v[i, pl.ds(f * 16, 16)] = jnp.zeros((16,), jnp.float32)
            return carry

        lax.fori_loop(jnp.int32(0), jnp.int32(160), fill, jnp.int32(0))

        def zero(off, ln):
            pltpu.sync_copy(stage_v.at[pl.ds(0, ln)],
                            acc_sh.at[pl.ds(off, ln)])

        _each_piece(zero)
        plsc.subcore_barrier()

        def start_gather(t, b):
            b = jnp.int32(b)
            pltpu.async_copy(p_hbm.at[ridx_v.at[t]], rows_v.at[b],
                             gsem.at[b])

        def wait_gather(t, b):
            b = jnp.int32(b)
            pltpu.make_async_copy(p_hbm.at[ridx_v.at[t]], rows_v.at[b],
                                  gsem.at[b]).wait()

        def start_scatter(t, b):
            b = jnp.int32(b)
            pltpu.async_copy(rows_v.at[b], acc_sh.at[cidx_v.at[t]],
                             ssem.at[b], add=True)

        def wait_scatter(b):
            b = jnp.int32(b)
            pltpu.make_async_copy(rows_v.at[b], acc_sh.at[cidx_v.at[jnp.int32(0)]],
                                  ssem.at[b]).wait()

        start_gather(jnp.int32(0), 0)

        def tri_body(u, carry):
            for k in range(5):
                t = u * 5 + k
                b = k
                nb = (k + 1) % 5

                @pl.when(t < nch)
                def _():
                    @pl.when(t + 1 < nch)
                    def _():
                        @pl.when(t >= 4)
                        def _():
                            wait_scatter(nb)

                        start_gather(t + 1, nb)

                    wait_gather(t, b)

                    def g_body(g, gcarry):
                        ewg = ew_v[pl.ds(t * _CK + g * 16, 16)]
                        for j in range(16):
                            e = g * 16 + j
                            wv = jnp.full((16,), ewg[j], jnp.float32)
                            for f in range(nvreg):
                                rows_v[b, e, pl.ds(f * 16, 16)] = (
                                    rows_v[b, e, pl.ds(f * 16, 16)] * wv)
                        return gcarry

                    lax.fori_loop(jnp.int32(0), jnp.int32(_CK // 16), g_body,
                                  jnp.int32(0))
                    start_scatter(t, b)

            return carry

        lax.fori_loop(jnp.int32(0), jnp.int32(-(-_CPT // 5)), tri_body,
                      jnp.int32(0))
        wait_scatter(0)
        wait_scatter(1)
        wait_scatter(2)
        wait_scatter(3)
        wait_scatter(4)
        plsc.subcore_barrier()

        def writeout(off, ln):
            pltpu.sync_copy(acc_sh.at[pl.ds(off, ln)],
                            stage_v.at[pl.ds(0, ln)])
            pltpu.sync_copy(stage_v.at[pl.ds(0, ln)],
                            out_hbm.at[pl.ds(c * _N + off, ln)])

        _each_piece(writeout)

    return _sc_conv


_sc_conv64 = _make_sc_conv(64)
_sc_conv16 = _make_sc_conv(16)


def _tc_first_body(deg_ref, x_ref, w1_ref, p1_ref, dinv_ref):
    deg = deg_ref[0] + deg_ref[1] + 1.0
    dinv = jnp.where(deg > 0, lax.rsqrt(deg), 0.0)
    dinv_ref[...] = dinv[:, None]
    mm = jnp.dot(x_ref[...], w1_ref[...], preferred_element_type=jnp.float32)
    p1_ref[...] = mm * dinv[:, None]


def _tc_mid_body(acc_ref, p1_ref, dinv_ref, b1_ref, w2_ref, p2_ref):
    dinv = dinv_ref[...]
    a = acc_ref[0] + acc_ref[1] + p1_ref[...]
    h = jnp.maximum(dinv * a + b1_ref[...], 0.0)
    p2_ref[...] = jnp.dot(h, w2_ref[...],
                          preferred_element_type=jnp.float32) * dinv


def _tc_last_body(acc_ref, p2_ref, dinv_ref, b2_ref, out_ref):
    a = acc_ref[0] + acc_ref[1] + p2_ref[...]
    out_ref[...] = dinv_ref[...] * a + b2_ref[...]


def kernel(x, edge_index, edge_weight, W1, b1, W2, b2):
    n, nfeat = x.shape
    nhid = W1.shape[1]
    ncls = W2.shape[1]

    row = edge_index[0].astype(jnp.int32).reshape(_NCHUNK, _CK)
    col = edge_index[1].astype(jnp.int32).reshape(_NCHUNK, _CK)
    ew = edge_weight.astype(jnp.float32)
    ew2d = ew.reshape(_NCHUNK, _CK)
    xf = x.astype(jnp.float32)

    deg_parts = _sc_degree(col, ew2d).reshape(_NC, n)

    p1, dinv = pl.pallas_call(
        _tc_first_body,
        out_shape=[jax.ShapeDtypeStruct((n, nhid), jnp.float32),
                   jax.ShapeDtypeStruct((n, 1), jnp.float32)],
    )(deg_parts, xf, W1.astype(jnp.float32))

    acc1 = _sc_conv64(row, col, ew, p1).reshape(_NC, n, nhid)

    p2 = pl.pallas_call(
        _tc_mid_body,
        out_shape=jax.ShapeDtypeStruct((n, ncls), jnp.float32),
    )(acc1, p1, dinv, b1.astype(jnp.float32), W2.astype(jnp.float32))

    acc2 = _sc_conv16(row, col, ew, p2).reshape(_NC, n, ncls)

    out = pl.pallas_call(
        _tc_last_body,
        out_shape=jax.ShapeDtypeStruct((n, ncls), jnp.float32),
    )(acc2, p2, dinv, b2.astype(jnp.float32))

    return out.astype(x.dtype if x.dtype == jnp.float64 else jnp.float64)
